# baseline (speedup 1.0000x reference)
import functools

import jax
import jax.numpy as jnp
import numpy as np
from jax import lax
from jax.experimental import pallas as pl
from jax.experimental.pallas import tpu as pltpu
from jax.experimental.pallas import tpu_sc as plsc

_B = 8
_T = 98432
_K = 256
_STRIDE = 128
_NUM_EMB = 1024
_D = 256
_N = 768
_FLAT = _B * _N
_GSTEPS = 8
_ROWS_STEP = _FLAT // _GSTEPS


def _tc_encode_body(xib_ref, cb_ref, idx_ref):
    g = pl.program_id(0)
    base = g * _ROWS_STEP
    a = xib_ref[pl.ds(base, _ROWS_STEP), :]
    b = xib_ref[pl.ds(base + _B, _ROWS_STEP), :]
    tokens = jnp.concatenate([a, b], axis=1)
    cb = cb_ref[...]
    cnorm = jnp.sum(cb * cb, axis=1, keepdims=True)
    prod = lax.dot_general(cb, tokens, (((1,), (1,)), ((), ())),
                           preferred_element_type=jnp.float32)
    scores = cnorm - 2.0 * prod
    m = jnp.min(scores, axis=0, keepdims=True)
    rows = lax.broadcasted_iota(jnp.int32, scores.shape, 0).astype(jnp.float32)
    idxf = jnp.min(jnp.where(scores <= m, rows, float(_NUM_EMB)), axis=0)
    idx_ref[0, 0, :] = idxf.astype(jnp.int32)


def _tc_encode(xib, cb):
    return pl.pallas_call(
        _tc_encode_body,
        grid=(_GSTEPS,),
        in_specs=[
            pl.BlockSpec(((_N + 1) * _B, _STRIDE), lambda i: (0, 0)),
            pl.BlockSpec((_NUM_EMB, _K), lambda i: (0, 0)),
        ],
        out_specs=pl.BlockSpec((1, 1, _ROWS_STEP), lambda i: (i, 0, 0)),
        out_shape=jax.ShapeDtypeStruct((_GSTEPS, 1, _ROWS_STEP), jnp.int32),
    )(xib, cb)


def _make_sc_gather():
    info = plsc.get_sparse_core_info()
    nc, ns = info.num_cores, info.num_subcores
    nw = nc * ns
    ppw = _N // nw
    rows_per_w = ppw * _B
    mesh = plsc.VectorSubcoreMesh(core_axis_name="c", subcore_axis_name="s")

    nchunk = 4
    chunk = rows_per_w // nchunk

    @functools.partial(
        pl.kernel,
        mesh=mesh,
        out_type=jax.ShapeDtypeStruct((_FLAT, _D), jnp.float32),
        scratch_types=[
            pltpu.VMEM((rows_per_w,), jnp.int32),
            pltpu.VMEM((rows_per_w, _D), jnp.float32),
            pltpu.VMEM((ppw, _D), jnp.float32),
            [pltpu.SemaphoreType.DMA] * nchunk,
            pltpu.SemaphoreType.DMA,
            pltpu.SemaphoreType.DMA,
        ],
    )
    def sc_gather(emb_hbm, idx_hbm, pe_hbm, out_hbm,
                  idx_v, rows_v, pe_v, gsems, sem_pe, sem_out):
        w = lax.axis_index("s") * nc + lax.axis_index("c")
        base = w * rows_per_w
        pltpu.sync_copy(idx_hbm.at[pl.ds(base, rows_per_w)], idx_v)
        cpe = pltpu.async_copy(pe_hbm.at[pl.ds(w * ppw, ppw)], pe_v, sem_pe)
        gathers = []
        for j in range(nchunk):
            sl = pl.ds(j * chunk, chunk)
            gathers.append(pltpu.async_copy(
                emb_hbm.at[idx_v.at[sl]], rows_v.at[sl], gsems[j]))
        cpe.wait()
        writebacks = []
        for j in range(nchunk):
            gathers[j].wait()

            @plsc.parallel_loop(j * chunk, (j + 1) * chunk, 1, unroll=2)
            def _(r):
                p = r // _B
                for c in range(_D // 16):
                    sl16 = pl.ds(c * 16, 16)
                    rows_v[r, sl16] = rows_v[r, sl16] + pe_v[p, sl16]

            sl = pl.ds(j * chunk, chunk)
            writebacks.append(pltpu.async_copy(
                rows_v.at[sl], out_hbm.at[pl.ds(base + j * chunk, chunk)],
                sem_out))
        for co in writebacks:
            co.wait()

    return sc_gather


def _positional_table():
    i = np.arange(_D // 2, dtype=np.float64)
    t = 1.0 / (10000.0 ** (2.0 * i / _D))
    pos = np.arange(_N, dtype=np.float64)[:, None] * _STRIDE
    s = np.sin(pos * t[None, :])
    c = np.cos(pos * t[None, :])
    pe = np.stack([s, c], axis=1)
    pe = np.transpose(pe, (0, 2, 1)).reshape(_N, _D)
    return jnp.asarray(pe.astype(np.float32))


def kernel(X, vq_codebook, emb_table):
    xib = jnp.transpose(X.reshape(_B, _N + 1, _STRIDE), (1, 0, 2))
    xib = xib.reshape((_N + 1) * _B, _STRIDE)
    idx3 = _tc_encode(xib, vq_codebook)
    sc_gather = _make_sc_gather()
    pe = _positional_table()
    out_flat = sc_gather(emb_table, idx3.reshape(_FLAT), pe)
    return jnp.transpose(out_flat.reshape(_N, _B, _D), (1, 0, 2))


# speedup vs baseline: 1.2146x; 1.2146x over previous
import functools

import jax
import jax.numpy as jnp
import numpy as np
from jax import lax
from jax.experimental import pallas as pl
from jax.experimental.pallas import tpu as pltpu
from jax.experimental.pallas import tpu_sc as plsc

_B = 8
_T = 98432
_K = 256
_STRIDE = 128
_NUM_EMB = 1024
_D = 256
_N = 768
_FLAT = _B * _N
_NCHUNK = 4


def _tc_encode_body(x_ref, cb_ref, idx_ref):
    x = x_ref[0]
    a = x[0:_N, :]
    b = x[1:_N + 1, :]
    tokens = jnp.concatenate([a, b], axis=1)
    cb = cb_ref[...]
    cnorm = jnp.sum(cb * cb, axis=1, keepdims=True)
    prod = lax.dot_general(cb, tokens, (((1,), (1,)), ((), ())),
                           preferred_element_type=jnp.float32)
    scores = cnorm - 2.0 * prod
    m = jnp.min(scores, axis=0, keepdims=True)
    rows = lax.broadcasted_iota(jnp.int32, scores.shape, 0).astype(jnp.float32)
    idxf = jnp.min(jnp.where(scores <= m, rows, float(_NUM_EMB)), axis=0)
    idx_ref[0, 0, :] = idxf.astype(jnp.int32)


def _tc_encode(xr, cb):
    return pl.pallas_call(
        _tc_encode_body,
        grid=(_B,),
        in_specs=[
            pl.BlockSpec((1, _N + 1, _STRIDE), lambda i: (i, 0, 0)),
            pl.BlockSpec((_NUM_EMB, _K), lambda i: (0, 0)),
        ],
        out_specs=pl.BlockSpec((1, 1, _N), lambda i: (i, 0, 0)),
        out_shape=jax.ShapeDtypeStruct((_B, 1, _N), jnp.int32),
    )(xr, cb)


def _make_sc_gather():
    info = plsc.get_sparse_core_info()
    nc, ns = info.num_cores, info.num_subcores
    nw = nc * ns
    rows_per_w = _FLAT // nw
    chunk = rows_per_w // _NCHUNK
    mesh = plsc.VectorSubcoreMesh(core_axis_name="c", subcore_axis_name="s")

    @functools.partial(
        pl.kernel,
        mesh=mesh,
        out_type=jax.ShapeDtypeStruct((_FLAT, _D), jnp.float32),
        scratch_types=[
            pltpu.VMEM((rows_per_w,), jnp.int32),
            pltpu.VMEM((rows_per_w, _D), jnp.float32),
            pltpu.VMEM((rows_per_w, _D), jnp.float32),
            [pltpu.SemaphoreType.DMA] * _NCHUNK,
            pltpu.SemaphoreType.DMA,
            pltpu.SemaphoreType.DMA,
        ],
    )
    def sc_gather(emb_hbm, idx_hbm, pe_hbm, out_hbm,
                  idx_v, rows_v, pe_v, gsems, sem_pe, sem_out):
        w = lax.axis_index("s") * nc + lax.axis_index("c")
        base = w * rows_per_w
        pltpu.sync_copy(idx_hbm.at[pl.ds(base, rows_per_w)], idx_v)
        gathers = []
        for j in range(_NCHUNK):
            sl = pl.ds(j * chunk, chunk)
            gathers.append(pltpu.async_copy(
                emb_hbm.at[idx_v.at[sl]], rows_v.at[sl], gsems[j]))
        cpe = pltpu.async_copy(
            pe_hbm.at[pl.ds((w % (_N // rows_per_w)) * rows_per_w, rows_per_w)],
            pe_v, sem_pe)
        cpe.wait()
        writebacks = []
        for j in range(_NCHUNK):
            gathers[j].wait()

            @plsc.parallel_loop(j * chunk, (j + 1) * chunk, 1, unroll=2)
            def _(r):
                for c in range(_D // 16):
                    sl16 = pl.ds(c * 16, 16)
                    rows_v[r, sl16] = rows_v[r, sl16] + pe_v[r, sl16]

            sl = pl.ds(j * chunk, chunk)
            writebacks.append(pltpu.async_copy(
                rows_v.at[sl], out_hbm.at[pl.ds(base + j * chunk, chunk)],
                sem_out))
        for co in writebacks:
            co.wait()

    return sc_gather


def _positional_table():
    i = np.arange(_D // 2, dtype=np.float64)
    t = 1.0 / (10000.0 ** (2.0 * i / _D))
    pos = np.arange(_N, dtype=np.float64)[:, None] * _STRIDE
    s = np.sin(pos * t[None, :])
    c = np.cos(pos * t[None, :])
    pe = np.stack([s, c], axis=1)
    pe = np.transpose(pe, (0, 2, 1)).reshape(_N, _D)
    return jnp.asarray(pe.astype(np.float32))


def kernel(X, vq_codebook, emb_table):
    xr = X.reshape(_B, _N + 1, _STRIDE)
    idx3 = _tc_encode(xr, vq_codebook)
    sc_gather = _make_sc_gather()
    pe = _positional_table()
    out_flat = sc_gather(emb_table, idx3.reshape(_FLAT), pe)
    return out_flat.reshape(_B, _N, _D)
